# baseline (speedup 1.0000x reference)
import jax
import jax.numpy as jnp
from jax.experimental import pallas as pl

N = 2048
K2 = 512
BR = 1024
NBLK = N // BR
F1, F2, F3 = 384, 640, 1152
BF = jnp.bfloat16


def _dot(a, b):
    return jnp.dot(a, b, preferred_element_type=jnp.float32)


def _dotb(a, b):
    return jnp.dot(a.astype(BF), b.astype(BF),
                   preferred_element_type=jnp.float32)


def _pad2(w, r, c):
    return jnp.pad(w, ((0, r - w.shape[0]), (0, c - w.shape[1])))


def _pad_row(b, c):
    return jnp.pad(b, (0, c - b.shape[0])).reshape(1, c)


def _pad_col(p, r):
    return jnp.pad(p, (0, r - p.shape[0])).reshape(r, 1)


def _pnorm(p):
    return jnp.maximum(jnp.sqrt(jnp.sum(p * p)), 1e-30)


def _emb_pair(cb, tv, twa, tba, twb, tbb, cwa, cba, cwb, cbb):
    ht = jax.nn.gelu(tv * twa[...] + tba[...])
    temb = _dotb(ht, twb[...]) + tbb[...]
    hc = jax.nn.gelu(_dot(cb, cwa[...]) + cba[...])
    cemb = _dotb(hc, cwb[...]) + cbb[...]
    return cemb, temb


def _dinv_body(a_ref, dv_ref, ab_ref):
    a = a_ref[...]
    dv_ref[...] = jax.lax.rsqrt(jnp.sum(a, axis=1, keepdims=True) + 1.0)
    ab_ref[...] = a.astype(BF)


def _h1s_body(a_ref, ab_ref, dvf_ref, dvb_ref, w1f_ref, w1b_ref,
              b1_ref, p1_ref, w2_ref, p2_ref,
              h1_ref, s1_ref, hv_ref):
    dvf = dvf_ref[...]
    dvb = dvb_ref[...]
    w1f = w1f_ref[...]
    w1b = w1b_ref[...]
    b1 = b1_ref[...]
    w1d = (dvf * w1f).astype(BF)
    w1db = dvb * w1b
    h1 = dvb * (jnp.dot(ab_ref[...], w1d, preferred_element_type=jnp.float32)
                + w1db) + b1
    h1_ref[...] = h1.astype(BF)
    a = a_ref[...]
    p1 = p1_ref[...]
    v1 = dvf * _dot(w1f, p1)
    v1b = dvb * _dot(w1b, p1)
    s1 = dvb * (_dot(a, v1) + v1b) + _dot(b1, p1)
    s1_ref[...] = s1 / _pnorm(p1)
    p2 = p2_ref[...] / _pnorm(p2_ref[...])
    w2v = _dot(w2_ref[...], p2)
    vh = dvf * _dot(w1f, w2v)
    vhb = dvb * _dot(w1b, w2v)
    hv_ref[...] = dvb * (_dot(a, vh) + vhb) + _dot(b1, w2v)


def _rank1_body(s_ref, st_ref, mo_ref, g_ref):
    s = s_ref[...]
    st = st_ref[...]
    ridx = jax.lax.broadcasted_iota(jnp.int32, (N, 1), 0)
    r = jnp.zeros((N, 1), jnp.float32)
    ch = 512
    for c in range(N // ch):
        col = st[:, c * ch:(c + 1) * ch]
        cidx = c * ch + jax.lax.broadcasted_iota(jnp.int32, (1, ch), 1)
        gt = (col > s) | ((col == s) & (cidx < ridx))
        r = r + jnp.sum(gt.astype(jnp.float32), axis=1, keepdims=True)
    m = (r < 1024).astype(jnp.float32)
    mo_ref[...] = m
    g_ref[...] = m * jnp.tanh(s)


def _deg1zt2_body(a_ref, mf_ref, mb_ref, g_ref, hv_ref, h1_ref, w2b_ref,
                  u_ref, z_ref, t2_ref):
    mb = mb_ref[...]
    g = g_ref[...]
    d = _dot(a_ref[...], mf_ref[...]) + mb
    u = mb * jax.lax.rsqrt(d + (1.0 - mb))
    u_ref[...] = u
    z_ref[...] = u * g * hv_ref[...]
    x = g * h1_ref[...].astype(jnp.float32)
    t2 = u * jnp.dot(x.astype(BF), w2b_ref[...],
                     preferred_element_type=jnp.float32)
    t2_ref[...] = t2.astype(BF)


def _gcn2_body(ab_ref, t2f_ref, t2b_ref, a_ref, zf_ref, zb_ref, u_ref,
               b2_ref, p2_ref, h2_ref, s2_ref):
    u = u_ref[...]
    t2b = t2b_ref[...].astype(jnp.float32)
    m = jnp.dot(ab_ref[...], t2f_ref[...],
                preferred_element_type=jnp.float32) + t2b
    b2 = b2_ref[...]
    h2 = u * m + b2
    h2_ref[...] = h2.astype(BF)
    sv = _dot(a_ref[...], zf_ref[...]) + zb_ref[...]
    p2 = p2_ref[...]
    b2v = _dot(b2, p2) / _pnorm(p2)
    s2_ref[...] = u * sv + b2v


def _rank2_body(s_ref, st_ref, m_ref, mt_ref, mo_ref, g_ref, oh_ref, oht_ref):
    s = s_ref[...]
    sm = jnp.where(m_ref[...] > 0, s, -3.0e38)
    smt = jnp.where(mt_ref[...] > 0, st_ref[...], -3.0e38)
    ridx = jax.lax.broadcasted_iota(jnp.int32, (N, 1), 0)
    cidx_f = jax.lax.broadcasted_iota(jnp.int32, (1, N), 1)
    ch = 512
    r = jnp.zeros((N, 1), jnp.float32)
    rt = jnp.zeros((1, N), jnp.float32)
    for c in range(N // ch):
        col = smt[:, c * ch:(c + 1) * ch]
        cidx = c * ch + jax.lax.broadcasted_iota(jnp.int32, (1, ch), 1)
        gt = (col > sm) | ((col == sm) & (cidx < ridx))
        r = r + jnp.sum(gt.astype(jnp.float32), axis=1, keepdims=True)
        rowv = sm[c * ch:(c + 1) * ch, :]
        rix = c * ch + jax.lax.broadcasted_iota(jnp.int32, (ch, 1), 0)
        gt2 = (rowv > smt) | ((rowv == smt) & (rix < cidx_f))
        rt = rt + jnp.sum(gt2.astype(jnp.float32), axis=0, keepdims=True)
    m = (r < K2).astype(jnp.float32)
    mo_ref[...] = m
    g_ref[...] = m * jnp.tanh(s)
    kidx = jax.lax.broadcasted_iota(jnp.int32, (1, K2), 1)
    oh_ref[...] = (r.astype(jnp.int32) == kidx).astype(BF)
    kidx2 = jax.lax.broadcasted_iota(jnp.int32, (K2, 1), 0)
    oht_ref[...] = (rt.astype(jnp.int32) == kidx2).astype(BF)


def _dega0s2_body(ab_ref, mf_ref, mb_ref, ohf_ref, ohb_ref, u_ref, o_ref):
    mb = mb_ref[...]
    ab = ab_ref[...]
    d = jnp.dot(ab, mf_ref[...].astype(BF),
                preferred_element_type=jnp.float32) + mb
    u_ref[...] = mb * jax.lax.rsqrt(d + (1.0 - mb))
    o = jnp.dot(ab, ohf_ref[...], preferred_element_type=jnp.float32)
    o_ref[...] = (o + ohb_ref[...].astype(jnp.float32)).astype(BF)


def _kA_body(oht_ref, h2_ref, a0s2_ref, u1_ref, u2_ref, gm2_ref,
             w3_ref, b3_ref, p3_ref,
             h3c_ref, a2c_ref, u1c_ref, u2c_ref, s3c_ref):
    oht = oht_ref[...]
    u1c = _dotb(oht, u1_ref[...])
    u2c = _dotb(oht, u2_ref[...])
    gm2c = _dotb(oht, gm2_ref[...])
    u1c_ref[...] = u1c
    u2c_ref[...] = u2c
    h2c = jnp.dot(oht, h2_ref[...], preferred_element_type=jnp.float32)
    a2cp = jnp.dot(oht, a0s2_ref[...], preferred_element_type=jnp.float32)
    a2c_ref[...] = a2cp.astype(BF)
    t3c = u2c * _dotb(gm2c * h2c, w3_ref[...])
    h3c = u2c * _dotb(a2cp, t3c) + b3_ref[...]
    h3c_ref[...] = h3c
    p = p3_ref[...]
    s3c_ref[...] = _dot(h3c, p) / _pnorm(p)


def _kB_body(s3c_ref, s3ct_ref, h3c_ref, a2c_ref, u1c_ref, u2c_ref,
             c_ref, cm_ref, t_ref,
             tw0a, tb0a, tw0b, tb0b, cw0a, cb0a, cw0b, cb0b,
             tw1a, tb1a, tw1b, tb1b, cw1a, cb1a, cw1b, cb1b,
             wu1_ref, bu1_ref, wu2_ref,
             tu2c_ref):
    cb = c_ref[...] * (1.0 - cm_ref[0, 0])
    tv = t_ref[0, 0]
    ce0, te0 = _emb_pair(cb, tv, tw0a, tb0a, tw0b, tb0b,
                         cw0a, cb0a, cw0b, cb0b)
    ce1, te1 = _emb_pair(cb, tv, tw1a, tb1a, tw1b, tb1b,
                         cw1a, cb1a, cw1b, cb1b)
    s = s3c_ref[...]
    st = s3ct_ref[...]
    ridx = jax.lax.broadcasted_iota(jnp.int32, (K2, 1), 0)
    cidx = jax.lax.broadcasted_iota(jnp.int32, (1, K2), 1)
    gt = (st > s) | ((st == s) & (cidx < ridx))
    r = jnp.sum(gt.astype(jnp.float32), axis=1, keepdims=True)
    m3c = (r < 6).astype(jnp.float32)
    gm3c = m3c * jnp.tanh(s)
    u2c = u2c_ref[...]
    x2uc = gm3c * h3c_ref[...] * ce0 + m3c * te0
    tu1c = u2c * _dotb(x2uc, wu1_ref[...])
    x2rc = u2c * jnp.dot(a2c_ref[...], tu1c.astype(BF),
                         preferred_element_type=jnp.float32) + bu1_ref[...]
    x1uc = x2rc * ce1 + te1
    tu2c = u1c_ref[...] * _dotb(x1uc, wu2_ref[...])
    tu2c_ref[...] = tu2c.astype(BF)


def _uy_body(a0s2_ref, tu2c_ref, u1_ref, bu2_ref, m1_ref, dv_ref,
             c_ref, cm_ref, t_ref,
             tw2a, tb2a, tw2b, tb2b, cw2a, cb2a, cw2b, cb2b,
             wu3_ref, o_ref):
    cb = c_ref[...] * (1.0 - cm_ref[0, 0])
    tv = t_ref[0, 0]
    ce2, te2 = _emb_pair(cb, tv, tw2a, tb2a, tw2b, tb2b,
                         cw2a, cb2a, cw2b, cb2b)
    x1r = u1_ref[...] * jnp.dot(a0s2_ref[...], tu2c_ref[...],
                                preferred_element_type=jnp.float32) \
        + bu2_ref[...]
    xu = m1_ref[...] * (x1r * ce2 + te2)
    y = jnp.dot(xu.astype(BF), wu3_ref[...],
                preferred_element_type=jnp.float32)
    o_ref[...] = (dv_ref[...] * y).astype(BF)


def _fin_body(ab_ref, yf_ref, yb_ref, dv_ref, b_ref, o_ref):
    m = jnp.dot(ab_ref[...], yf_ref[...], preferred_element_type=jnp.float32)
    o_ref[...] = dv_ref[...] * (m + yb_ref[...].astype(jnp.float32)) \
        + b_ref[...]


def _vspec(bm):
    return pl.BlockSpec((bm, 1), lambda i: (i, 0))


def _fix(shape):
    return pl.BlockSpec(shape, lambda i: (0, 0))


def _rowspec(bm, n):
    return pl.BlockSpec((bm, n), lambda i: (i, 0))


def _sds(shape, dtype=jnp.float32):
    return jax.ShapeDtypeStruct(shape, dtype)


def _run_dinv(a):
    return pl.pallas_call(
        _dinv_body, grid=(NBLK,),
        in_specs=[_rowspec(BR, N)],
        out_specs=(_vspec(BR), _rowspec(BR, N)),
        out_shape=(_sds((N, 1)), _sds((N, N), BF)),
    )(a)


def _run_h1s(a, ab, dv, w1, w1b, b1, p1, w2, p2):
    return pl.pallas_call(
        _h1s_body, grid=(NBLK,),
        in_specs=[_rowspec(BR, N), _rowspec(BR, N), _fix((N, 1)),
                  _vspec(BR), _fix((N, F1)), _rowspec(BR, F1),
                  _fix((1, F1)), _fix((F1, 1)), _fix((F1, F2)),
                  _fix((F2, 1))],
        out_specs=(_rowspec(BR, F1), _vspec(BR), _vspec(BR)),
        out_shape=(_sds((N, F1), BF), _sds((N, 1)), _sds((N, 1))),
    )(a, ab, dv, dv, w1, w1, b1, p1, w2, p2)


def _run_rank1(s):
    return pl.pallas_call(
        _rank1_body,
        out_shape=(_sds((N, 1)), _sds((N, 1))),
    )(s, s.reshape(1, N))


def _run_deg1zt2(a, m, gm, hv, h1, w2b):
    return pl.pallas_call(
        _deg1zt2_body, grid=(NBLK,),
        in_specs=[_rowspec(BR, N), _fix((N, 1)), _vspec(BR), _vspec(BR),
                  _vspec(BR), _rowspec(BR, F1), _fix((F1, F2))],
        out_specs=(_vspec(BR), _vspec(BR), _rowspec(BR, F2)),
        out_shape=(_sds((N, 1)), _sds((N, 1)), _sds((N, F2), BF)),
    )(a, m, m, gm, hv, h1, w2b)


def _run_gcn2(ab, t2, a, z, u, b2, p2):
    return pl.pallas_call(
        _gcn2_body, grid=(NBLK,),
        in_specs=[_rowspec(BR, N), _fix((N, F2)), _rowspec(BR, F2),
                  _rowspec(BR, N), _fix((N, 1)), _vspec(BR), _vspec(BR),
                  _fix((1, F2)), _fix((F2, 1))],
        out_specs=(_rowspec(BR, F2), _vspec(BR)),
        out_shape=(_sds((N, F2), BF), _sds((N, 1))),
    )(ab, t2, t2, a, z, z, u, b2, p2)


def _run_rank2(s, mprev):
    return pl.pallas_call(
        _rank2_body,
        out_shape=(_sds((N, 1)), _sds((N, 1)),
                   _sds((N, K2), BF), _sds((K2, N), BF)),
    )(s, s.reshape(1, N), mprev, mprev.reshape(1, N))


def _run_dega0s2(ab, m, oh):
    return pl.pallas_call(
        _dega0s2_body, grid=(NBLK,),
        in_specs=[_rowspec(BR, N), _fix((N, 1)), _vspec(BR),
                  _fix((N, K2)), _rowspec(BR, K2)],
        out_specs=(_vspec(BR), _rowspec(BR, K2)),
        out_shape=(_sds((N, 1)), _sds((N, K2), BF)),
    )(ab, m, m, oh, oh)


def _run_kA(oht, h2, a0s2, u1, u2, gm2, w3, b3, p3):
    return pl.pallas_call(
        _kA_body,
        out_shape=(_sds((K2, F3)), _sds((K2, K2), BF), _sds((K2, 1)),
                   _sds((K2, 1)), _sds((K2, 1))),
    )(oht, h2, a0s2, u1, u2, gm2, w3, b3, p3)


def _run_kB(s3c, h3c, a2c, u1c, u2c, cmt, emb01, wu1, bu1, wu2):
    return pl.pallas_call(
        _kB_body,
        out_shape=_sds((K2, F1), BF),
    )(s3c, s3c.reshape(1, K2), h3c, a2c, u1c, u2c,
      *cmt, *emb01, wu1, bu1, wu2)


def _run_uy(a0s2, tu2c, u1, bu2, m1, dv, cmt, emb2, wu3b):
    return pl.pallas_call(
        _uy_body, grid=(NBLK,),
        in_specs=[_rowspec(BR, K2), _fix((K2, F1)), _vspec(BR),
                  _fix((1, F1)), _vspec(BR), _vspec(BR),
                  _fix((1, 16)), _fix((1, 1)), _fix((1, 1)),
                  _fix((1, F1)), _fix((1, F1)), _fix((F1, F1)), _fix((1, F1)),
                  _fix((16, F1)), _fix((1, F1)), _fix((F1, F1)), _fix((1, F1)),
                  _fix((F1, N))],
        out_specs=_rowspec(BR, N),
        out_shape=_sds((N, N), BF),
    )(a0s2, tu2c, u1, bu2, m1, dv, *cmt, *emb2, wu3b)


def _run_final(ab, y, dv, b):
    bm = 1024
    return pl.pallas_call(
        _fin_body, grid=(N // bm,),
        in_specs=[_rowspec(bm, N), _fix((N, N)), _rowspec(bm, N),
                  _vspec(bm), _fix((1, N))],
        out_specs=_rowspec(bm, N),
        out_shape=_sds((N, N)),
    )(ab, y, y, dv, b)


def kernel(x, c, t, context_mask, W1, b1, p1, W2, b2, p2, W3, b3, p3,
           Wu1, bu1, Wu2, bu2, Wu3, bu3,
           TW0a, Tb0a, TW0b, Tb0b, CW0a, Cb0a, CW0b, Cb0b,
           TW1a, Tb1a, TW1b, Tb1b, CW1a, Cb1a, CW1b, Cb1b,
           TW2a, Tb2a, TW2b, Tb2b, CW2a, Cb2a, CW2b, Cb2b):
    a0 = x[0, 0]

    w1p = _pad2(W1, N, F1)
    b1p = _pad_row(b1, F1)
    p1p = _pad_col(p1, F1)
    w2p = _pad2(W2, F1, F2)
    b2p = _pad_row(b2, F2)
    p2p = _pad_col(p2, F2)
    w3pb = _pad2(W3, F2, F3).astype(BF)
    b3p = _pad_row(b3, F3)
    p3p = _pad_col(p3, F3)
    wu1pb = _pad2(Wu1, F3, F2).astype(BF)
    bu1p = _pad_row(bu1, F2)
    wu2pb = _pad2(Wu2, F2, F1).astype(BF)
    bu2p = _pad_row(bu2, F1)
    wu3pb = _pad2(Wu3, F1, N).astype(BF)
    bu3p = _pad_row(bu3, N)

    cmt = [_pad2(c, 1, 16), context_mask.reshape(1, 1), t.reshape(1, 1)]
    emb01 = [
        _pad2(TW0a, 1, F3), _pad_row(Tb0a, F3), _pad2(TW0b, F3, F3).astype(BF), _pad_row(Tb0b, F3),
        _pad2(CW0a, 16, F3), _pad_row(Cb0a, F3), _pad2(CW0b, F3, F3).astype(BF), _pad_row(Cb0b, F3),
        _pad2(TW1a, 1, F2), _pad_row(Tb1a, F2), _pad2(TW1b, F2, F2).astype(BF), _pad_row(Tb1b, F2),
        _pad2(CW1a, 16, F2), _pad_row(Cb1a, F2), _pad2(CW1b, F2, F2).astype(BF), _pad_row(Cb1b, F2),
    ]
    emb2 = [
        _pad2(TW2a, 1, F1), _pad_row(Tb2a, F1), _pad2(TW2b, F1, F1).astype(BF), _pad_row(Tb2b, F1),
        _pad2(CW2a, 16, F1), _pad_row(Cb2a, F1), _pad2(CW2b, F1, F1).astype(BF), _pad_row(Cb2b, F1),
    ]

    dv, a0b = _run_dinv(a0)
    h1b, s1, hv = _run_h1s(a0, a0b, dv, w1p, w1p, b1p, p1p, w2p, p2p)

    m1, gm1 = _run_rank1(s1)
    u1, z, t2b = _run_deg1zt2(a0, m1, gm1, hv, h1b, w2p.astype(BF))
    h2b, s2 = _run_gcn2(a0b, t2b, a0, z, u1, b2p, p2p)

    m2, gm2, oh2b, oh2tb = _run_rank2(s2, m1)
    u2, a0s2b = _run_dega0s2(a0b, m2, oh2b)
    h3c, a2cb, u1c, u2c, s3c = _run_kA(oh2tb, h2b, a0s2b, u1, u2, gm2,
                                       w3pb, b3p, p3p)
    tu2cb = _run_kB(s3c, h3c, a2cb, u1c, u2c, cmt, emb01,
                    wu1pb, bu1p, wu2pb)
    yd = _run_uy(a0s2b, tu2cb, u1, bu2p, m1, dv, cmt, emb2, wu3pb)
    return _run_final(a0b, yd, dv, bu3p)


# speedup vs baseline: 1.0414x; 1.0414x over previous
import jax
import jax.numpy as jnp
from jax.experimental import pallas as pl

N = 2048
K2 = 512
BR = 512
NBLK = N // BR
F1, F2, F3 = 384, 640, 1152
BF = jnp.bfloat16


def _dot(a, b):
    return jnp.dot(a, b, preferred_element_type=jnp.float32)


def _dotb(a, b):
    return jnp.dot(a.astype(BF), b.astype(BF),
                   preferred_element_type=jnp.float32)


def _pad2(w, r, c):
    return jnp.pad(w, ((0, r - w.shape[0]), (0, c - w.shape[1])))


def _pad_row(b, c):
    return jnp.pad(b, (0, c - b.shape[0])).reshape(1, c)


def _pad_col(p, r):
    return jnp.pad(p, (0, r - p.shape[0])).reshape(r, 1)


def _pnorm(p):
    return jnp.maximum(jnp.sqrt(jnp.sum(p * p)), 1e-30)


def _emb_pair(cb, tv, twa, tba, twb, tbb, cwa, cba, cwb, cbb):
    ht = jax.nn.gelu(tv * twa[...] + tba[...])
    temb = _dotb(ht, twb[...]) + tbb[...]
    hc = jax.nn.gelu(_dot(cb, cwa[...]) + cba[...])
    cemb = _dotb(hc, cwb[...]) + cbb[...]
    return cemb, temb


def _dinv_body(a_ref, dv_ref, ab_ref):
    a = a_ref[...]
    dv_ref[...] = jax.lax.rsqrt(jnp.sum(a, axis=1, keepdims=True) + 1.0)
    ab_ref[...] = a.astype(BF)


def _h1s_body(a_ref, ab_ref, dvf_ref, dvb_ref, w1f_ref, w1b_ref,
              b1_ref, p1_ref, w2_ref, p2_ref,
              h1_ref, s1_ref, hv_ref):
    dvf = dvf_ref[...]
    dvb = dvb_ref[...]
    w1f = w1f_ref[...]
    w1b = w1b_ref[...]
    b1 = b1_ref[...]
    w1d = (dvf * w1f).astype(BF)
    w1db = dvb * w1b
    h1 = dvb * (jnp.dot(ab_ref[...], w1d, preferred_element_type=jnp.float32)
                + w1db) + b1
    h1_ref[...] = h1.astype(BF)
    a = a_ref[...]
    p1 = p1_ref[...]
    v1 = dvf * _dot(w1f, p1)
    v1b = dvb * _dot(w1b, p1)
    s1 = dvb * (_dot(a, v1) + v1b) + _dot(b1, p1)
    s1_ref[...] = s1 / _pnorm(p1)
    p2 = p2_ref[...] / _pnorm(p2_ref[...])
    w2v = _dot(w2_ref[...], p2)
    vh = dvf * _dot(w1f, w2v)
    vhb = dvb * _dot(w1b, w2v)
    hv_ref[...] = dvb * (_dot(a, vh) + vhb) + _dot(b1, w2v)


def _rank1_body(s_ref, st_ref, mo_ref, g_ref):
    s = s_ref[...]
    st = st_ref[...]
    ridx = jax.lax.broadcasted_iota(jnp.int32, (N, 1), 0)
    r = jnp.zeros((N, 1), jnp.float32)
    ch = 512
    for c in range(N // ch):
        col = st[:, c * ch:(c + 1) * ch]
        cidx = c * ch + jax.lax.broadcasted_iota(jnp.int32, (1, ch), 1)
        gt = (col > s) | ((col == s) & (cidx < ridx))
        r = r + jnp.sum(gt.astype(jnp.float32), axis=1, keepdims=True)
    m = (r < 1024).astype(jnp.float32)
    mo_ref[...] = m
    g_ref[...] = m * jnp.tanh(s)


def _deg1zt2_body(a_ref, mf_ref, mb_ref, g_ref, hv_ref, h1_ref, w2b_ref,
                  u_ref, z_ref, t2_ref):
    mb = mb_ref[...]
    g = g_ref[...]
    d = _dot(a_ref[...], mf_ref[...]) + mb
    u = mb * jax.lax.rsqrt(d + (1.0 - mb))
    u_ref[...] = u
    z_ref[...] = u * g * hv_ref[...]
    x = g * h1_ref[...].astype(jnp.float32)
    t2 = u * jnp.dot(x.astype(BF), w2b_ref[...],
                     preferred_element_type=jnp.float32)
    t2_ref[...] = t2.astype(BF)


def _gcn2_body(ab_ref, t2f_ref, t2b_ref, a_ref, zf_ref, zb_ref, u_ref,
               b2_ref, p2_ref, h2_ref, s2_ref):
    u = u_ref[...]
    t2b = t2b_ref[...].astype(jnp.float32)
    m = jnp.dot(ab_ref[...], t2f_ref[...],
                preferred_element_type=jnp.float32) + t2b
    b2 = b2_ref[...]
    h2 = u * m + b2
    h2_ref[...] = h2.astype(BF)
    sv = _dot(a_ref[...], zf_ref[...]) + zb_ref[...]
    p2 = p2_ref[...]
    b2v = _dot(b2, p2) / _pnorm(p2)
    s2_ref[...] = u * sv + b2v


def _rank2_body(s_ref, st_ref, m_ref, mt_ref, mo_ref, g_ref, oh_ref, oht_ref):
    s = s_ref[...]
    sm = jnp.where(m_ref[...] > 0, s, -3.0e38)
    smt = jnp.where(mt_ref[...] > 0, st_ref[...], -3.0e38)
    ridx = jax.lax.broadcasted_iota(jnp.int32, (N, 1), 0)
    cidx_f = jax.lax.broadcasted_iota(jnp.int32, (1, N), 1)
    ch = 512
    r = jnp.zeros((N, 1), jnp.float32)
    rt = jnp.zeros((1, N), jnp.float32)
    for c in range(N // ch):
        col = smt[:, c * ch:(c + 1) * ch]
        cidx = c * ch + jax.lax.broadcasted_iota(jnp.int32, (1, ch), 1)
        gt = (col > sm) | ((col == sm) & (cidx < ridx))
        r = r + jnp.sum(gt.astype(jnp.float32), axis=1, keepdims=True)
        rowv = sm[c * ch:(c + 1) * ch, :]
        rix = c * ch + jax.lax.broadcasted_iota(jnp.int32, (ch, 1), 0)
        gt2 = (rowv > smt) | ((rowv == smt) & (rix < cidx_f))
        rt = rt + jnp.sum(gt2.astype(jnp.float32), axis=0, keepdims=True)
    m = (r < K2).astype(jnp.float32)
    mo_ref[...] = m
    g_ref[...] = m * jnp.tanh(s)
    kidx = jax.lax.broadcasted_iota(jnp.int32, (1, K2), 1)
    oh_ref[...] = (r.astype(jnp.int32) == kidx).astype(BF)
    kidx2 = jax.lax.broadcasted_iota(jnp.int32, (K2, 1), 0)
    oht_ref[...] = (rt.astype(jnp.int32) == kidx2).astype(BF)


def _dega0s2_body(ab_ref, mf_ref, mb_ref, ohf_ref, ohb_ref, u_ref, o_ref):
    mb = mb_ref[...]
    ab = ab_ref[...]
    d = jnp.dot(ab, mf_ref[...].astype(BF),
                preferred_element_type=jnp.float32) + mb
    u_ref[...] = mb * jax.lax.rsqrt(d + (1.0 - mb))
    o = jnp.dot(ab, ohf_ref[...], preferred_element_type=jnp.float32)
    o_ref[...] = (o + ohb_ref[...].astype(jnp.float32)).astype(BF)


def _kAB_body(oht_ref, h2_ref, a0s2_ref, u1_ref, u2_ref, gm2_ref,
              w3_ref, b3_ref, p3_ref,
              c_ref, cm_ref, t_ref,
              tw0a, tb0a, tw0b, tb0b, cw0a, cb0a, cw0b, cb0b,
              tw1a, tb1a, tw1b, tb1b, cw1a, cb1a, cw1b, cb1b,
              wu1_ref, bu1_ref, wu2_ref,
              tu2c_ref):
    oht = oht_ref[...]
    u1c = _dotb(oht, u1_ref[...])
    u2c = _dotb(oht, u2_ref[...])
    gm2c = _dotb(oht, gm2_ref[...])
    h2c = jnp.dot(oht, h2_ref[...], preferred_element_type=jnp.float32)
    a2cp = jnp.dot(oht, a0s2_ref[...], preferred_element_type=jnp.float32)
    t3c = u2c * _dotb(gm2c * h2c, w3_ref[...])
    h3c = u2c * _dotb(a2cp.astype(BF), t3c) + b3_ref[...]
    p = p3_ref[...]
    s = _dot(h3c, p) / _pnorm(p)
    ridx = jax.lax.broadcasted_iota(jnp.int32, (K2, 1), 0)
    sel = jnp.zeros((K2, 1), jnp.bool_)
    sc = s
    for _ in range(6):
        mx = jnp.max(sc)
        cand = jnp.where(sc == mx, ridx, K2)
        imin = jnp.min(cand)
        hit = ridx == imin
        sel = sel | hit
        sc = jnp.where(hit, -3.0e38, sc)
    m3c = sel.astype(jnp.float32)
    gm3c = m3c * jnp.tanh(s)
    cb = c_ref[...] * (1.0 - cm_ref[0, 0])
    tv = t_ref[0, 0]
    ce0, te0 = _emb_pair(cb, tv, tw0a, tb0a, tw0b, tb0b,
                         cw0a, cb0a, cw0b, cb0b)
    ce1, te1 = _emb_pair(cb, tv, tw1a, tb1a, tw1b, tb1b,
                         cw1a, cb1a, cw1b, cb1b)
    x2uc = gm3c * h3c * ce0 + m3c * te0
    tu1c = u2c * _dotb(x2uc, wu1_ref[...])
    x2rc = u2c * jnp.dot(a2cp.astype(BF), tu1c.astype(BF),
                         preferred_element_type=jnp.float32) + bu1_ref[...]
    x1uc = x2rc * ce1 + te1
    tu2c = u1c * _dotb(x1uc, wu2_ref[...])
    tu2c_ref[...] = tu2c.astype(BF)


def _uy_body(a0s2_ref, tu2c_ref, u1_ref, bu2_ref, m1_ref, dv_ref,
             c_ref, cm_ref, t_ref,
             tw2a, tb2a, tw2b, tb2b, cw2a, cb2a, cw2b, cb2b,
             wu3_ref, o_ref):
    cb = c_ref[...] * (1.0 - cm_ref[0, 0])
    tv = t_ref[0, 0]
    ce2, te2 = _emb_pair(cb, tv, tw2a, tb2a, tw2b, tb2b,
                         cw2a, cb2a, cw2b, cb2b)
    x1r = u1_ref[...] * jnp.dot(a0s2_ref[...], tu2c_ref[...],
                                preferred_element_type=jnp.float32) \
        + bu2_ref[...]
    xu = m1_ref[...] * (x1r * ce2 + te2)
    y = jnp.dot(xu.astype(BF), wu3_ref[...],
                preferred_element_type=jnp.float32)
    o_ref[...] = (dv_ref[...] * y).astype(BF)


def _fin_body(ab_ref, yf_ref, yb_ref, dv_ref, b_ref, o_ref):
    m = jnp.dot(ab_ref[...], yf_ref[...], preferred_element_type=jnp.float32)
    o_ref[...] = dv_ref[...] * (m + yb_ref[...].astype(jnp.float32)) \
        + b_ref[...]


def _vspec(bm):
    return pl.BlockSpec((bm, 1), lambda i: (i, 0))


def _fix(shape):
    return pl.BlockSpec(shape, lambda i: (0, 0))


def _rowspec(bm, n):
    return pl.BlockSpec((bm, n), lambda i: (i, 0))


def _sds(shape, dtype=jnp.float32):
    return jax.ShapeDtypeStruct(shape, dtype)


def _run_dinv(a):
    return pl.pallas_call(
        _dinv_body, grid=(NBLK,),
        in_specs=[_rowspec(BR, N)],
        out_specs=(_vspec(BR), _rowspec(BR, N)),
        out_shape=(_sds((N, 1)), _sds((N, N), BF)),
    )(a)


def _run_h1s(a, ab, dv, w1, w1b, b1, p1, w2, p2):
    return pl.pallas_call(
        _h1s_body, grid=(NBLK,),
        in_specs=[_rowspec(BR, N), _rowspec(BR, N), _fix((N, 1)),
                  _vspec(BR), _fix((N, F1)), _rowspec(BR, F1),
                  _fix((1, F1)), _fix((F1, 1)), _fix((F1, F2)),
                  _fix((F2, 1))],
        out_specs=(_rowspec(BR, F1), _vspec(BR), _vspec(BR)),
        out_shape=(_sds((N, F1), BF), _sds((N, 1)), _sds((N, 1))),
    )(a, ab, dv, dv, w1, w1, b1, p1, w2, p2)


def _run_rank1(s):
    return pl.pallas_call(
        _rank1_body,
        out_shape=(_sds((N, 1)), _sds((N, 1))),
    )(s, s.reshape(1, N))


def _run_deg1zt2(a, m, gm, hv, h1, w2b):
    return pl.pallas_call(
        _deg1zt2_body, grid=(NBLK,),
        in_specs=[_rowspec(BR, N), _fix((N, 1)), _vspec(BR), _vspec(BR),
                  _vspec(BR), _rowspec(BR, F1), _fix((F1, F2))],
        out_specs=(_vspec(BR), _vspec(BR), _rowspec(BR, F2)),
        out_shape=(_sds((N, 1)), _sds((N, 1)), _sds((N, F2), BF)),
    )(a, m, m, gm, hv, h1, w2b)


def _run_gcn2(ab, t2, a, z, u, b2, p2):
    return pl.pallas_call(
        _gcn2_body, grid=(NBLK,),
        in_specs=[_rowspec(BR, N), _fix((N, F2)), _rowspec(BR, F2),
                  _rowspec(BR, N), _fix((N, 1)), _vspec(BR), _vspec(BR),
                  _fix((1, F2)), _fix((F2, 1))],
        out_specs=(_rowspec(BR, F2), _vspec(BR)),
        out_shape=(_sds((N, F2), BF), _sds((N, 1))),
    )(ab, t2, t2, a, z, z, u, b2, p2)


def _run_rank2(s, mprev):
    return pl.pallas_call(
        _rank2_body,
        out_shape=(_sds((N, 1)), _sds((N, 1)),
                   _sds((N, K2), BF), _sds((K2, N), BF)),
    )(s, s.reshape(1, N), mprev, mprev.reshape(1, N))


def _run_dega0s2(ab, m, oh):
    return pl.pallas_call(
        _dega0s2_body, grid=(NBLK,),
        in_specs=[_rowspec(BR, N), _fix((N, 1)), _vspec(BR),
                  _fix((N, K2)), _rowspec(BR, K2)],
        out_specs=(_vspec(BR), _rowspec(BR, K2)),
        out_shape=(_sds((N, 1)), _sds((N, K2), BF)),
    )(ab, m, m, oh, oh)


def _run_kAB(oht, h2, a0s2, u1, u2, gm2, w3, b3, p3, cmt, emb01,
             wu1, bu1, wu2):
    return pl.pallas_call(
        _kAB_body,
        out_shape=_sds((K2, F1), BF),
    )(oht, h2, a0s2, u1, u2, gm2, w3, b3, p3, *cmt, *emb01, wu1, bu1, wu2)


def _run_uy(a0s2, tu2c, u1, bu2, m1, dv, cmt, emb2, wu3b):
    return pl.pallas_call(
        _uy_body, grid=(NBLK,),
        in_specs=[_rowspec(BR, K2), _fix((K2, F1)), _vspec(BR),
                  _fix((1, F1)), _vspec(BR), _vspec(BR),
                  _fix((1, 16)), _fix((1, 1)), _fix((1, 1)),
                  _fix((1, F1)), _fix((1, F1)), _fix((F1, F1)), _fix((1, F1)),
                  _fix((16, F1)), _fix((1, F1)), _fix((F1, F1)), _fix((1, F1)),
                  _fix((F1, N))],
        out_specs=_rowspec(BR, N),
        out_shape=_sds((N, N), BF),
    )(a0s2, tu2c, u1, bu2, m1, dv, *cmt, *emb2, wu3b)


def _run_final(ab, y, dv, b):
    bm = 512
    return pl.pallas_call(
        _fin_body, grid=(N // bm,),
        in_specs=[_rowspec(bm, N), _fix((N, N)), _rowspec(bm, N),
                  _vspec(bm), _fix((1, N))],
        out_specs=_rowspec(bm, N),
        out_shape=_sds((N, N)),
    )(ab, y, y, dv, b)


def kernel(x, c, t, context_mask, W1, b1, p1, W2, b2, p2, W3, b3, p3,
           Wu1, bu1, Wu2, bu2, Wu3, bu3,
           TW0a, Tb0a, TW0b, Tb0b, CW0a, Cb0a, CW0b, Cb0b,
           TW1a, Tb1a, TW1b, Tb1b, CW1a, Cb1a, CW1b, Cb1b,
           TW2a, Tb2a, TW2b, Tb2b, CW2a, Cb2a, CW2b, Cb2b):
    a0 = x[0, 0]

    w1p = _pad2(W1, N, F1)
    b1p = _pad_row(b1, F1)
    p1p = _pad_col(p1, F1)
    w2p = _pad2(W2, F1, F2)
    b2p = _pad_row(b2, F2)
    p2p = _pad_col(p2, F2)
    w3pb = _pad2(W3, F2, F3).astype(BF)
    b3p = _pad_row(b3, F3)
    p3p = _pad_col(p3, F3)
    wu1pb = _pad2(Wu1, F3, F2).astype(BF)
    bu1p = _pad_row(bu1, F2)
    wu2pb = _pad2(Wu2, F2, F1).astype(BF)
    bu2p = _pad_row(bu2, F1)
    wu3pb = _pad2(Wu3, F1, N).astype(BF)
    bu3p = _pad_row(bu3, N)

    cmt = [_pad2(c, 1, 16), context_mask.reshape(1, 1), t.reshape(1, 1)]
    emb01 = [
        _pad2(TW0a, 1, F3), _pad_row(Tb0a, F3), _pad2(TW0b, F3, F3).astype(BF), _pad_row(Tb0b, F3),
        _pad2(CW0a, 16, F3), _pad_row(Cb0a, F3), _pad2(CW0b, F3, F3).astype(BF), _pad_row(Cb0b, F3),
        _pad2(TW1a, 1, F2), _pad_row(Tb1a, F2), _pad2(TW1b, F2, F2).astype(BF), _pad_row(Tb1b, F2),
        _pad2(CW1a, 16, F2), _pad_row(Cb1a, F2), _pad2(CW1b, F2, F2).astype(BF), _pad_row(Cb1b, F2),
    ]
    emb2 = [
        _pad2(TW2a, 1, F1), _pad_row(Tb2a, F1), _pad2(TW2b, F1, F1).astype(BF), _pad_row(Tb2b, F1),
        _pad2(CW2a, 16, F1), _pad_row(Cb2a, F1), _pad2(CW2b, F1, F1).astype(BF), _pad_row(Cb2b, F1),
    ]

    dv, a0b = _run_dinv(a0)
    h1b, s1, hv = _run_h1s(a0, a0b, dv, w1p, w1p, b1p, p1p, w2p, p2p)

    m1, gm1 = _run_rank1(s1)
    u1, z, t2b = _run_deg1zt2(a0, m1, gm1, hv, h1b, w2p.astype(BF))
    h2b, s2 = _run_gcn2(a0b, t2b, a0, z, u1, b2p, p2p)

    m2, gm2, oh2b, oh2tb = _run_rank2(s2, m1)
    u2, a0s2b = _run_dega0s2(a0b, m2, oh2b)
    tu2cb = _run_kAB(oh2tb, h2b, a0s2b, u1, u2, gm2, w3pb, b3p, p3p,
                     cmt, emb01, wu1pb, bu1p, wu2pb)
    yd = _run_uy(a0s2b, tu2cb, u1, bu2p, m1, dv, cmt, emb2, wu3pb)
    return _run_final(a0b, yd, dv, bu3p)
